# baseline (speedup 1.0000x reference)
import functools

import jax
import jax.numpy as jnp
from jax import lax
from jax.experimental import pallas as pl
from jax.experimental.pallas import tpu as pltpu
from jax.experimental.pallas import tpu_sc as plsc

DIM = 64
BATCH = 16384
NC = 2
NS = 16
NW = NC * NS
CPW = BATCH // NW
CHUNK = 128
NCHUNK = CPW // CHUNK
LANES = 16


def _score_chunk(es_v, ep_v, eo_v, part_f, out_v, out_base):
    iota16 = lax.iota(jnp.int32, LANES) * LANES

    def group(g, _):
        rowb = g * LANES

        def row(rr, _):
            r = rowb + rr
            v = (es_v[r, pl.ds(0, LANES)]
                 * ep_v[r, pl.ds(0, LANES)]
                 * eo_v[r, pl.ds(0, LANES)])
            for q in range(1, DIM // LANES):
                sl = pl.ds(q * LANES, LANES)
                v = v + es_v[r, sl] * ep_v[r, sl] * eo_v[r, sl]
            part_f[pl.ds(rr * LANES, LANES)] = v
            return 0

        lax.fori_loop(0, LANES, row, 0)

        def red(j, acc):
            return acc + plsc.load_gather(part_f, [iota16 + j])

        acc = lax.fori_loop(0, LANES, red, jnp.zeros((LANES,), jnp.float32))
        out_v[pl.ds(out_base + g * LANES, LANES)] = acc
        return 0

    lax.fori_loop(0, CHUNK // LANES, group, 0)


def _body(human, gmf, hm, rm, tm, hf, rf, tf,
          neg_o, dm_o, df_o,
          hmv, rmv, tmv, hfv, rfv, tfv,
          es_v, ep_v, eo_v, part_f,
          dm_v, df_v, ng_v,
          sem):
    wid = lax.axis_index("s") * NC + lax.axis_index("c")
    base = pl.multiple_of(wid * CPW, CPW)

    pltpu.sync_copy(hm.at[pl.ds(base, CPW)], hmv)
    pltpu.sync_copy(rm.at[pl.ds(base, CPW)], rmv)
    pltpu.sync_copy(tm.at[pl.ds(base, CPW)], tmv)
    pltpu.sync_copy(hf.at[pl.ds(base, CPW)], hfv)
    pltpu.sync_copy(rf.at[pl.ds(base, CPW)], rfv)
    pltpu.sync_copy(tf.at[pl.ds(base, CPW)], tfv)

    for (hv, rv, tv, out_v) in ((hmv, rmv, tmv, dm_v), (hfv, rfv, tfv, df_v)):
        for c in range(NCHUNK):
            sl = pl.ds(c * CHUNK, CHUNK)
            cp1 = pltpu.make_async_copy(human.at[hv.at[sl]], es_v, sem)
            cp2 = pltpu.make_async_copy(gmf.at[rv.at[sl]], ep_v, sem)
            cp3 = pltpu.make_async_copy(gmf.at[tv.at[sl]], eo_v, sem)
            cp1.start()
            cp2.start()
            cp3.start()
            cp1.wait()
            cp2.wait()
            cp3.wait()
            _score_chunk(es_v, ep_v, eo_v, part_f, out_v, c * CHUNK)

    for i in range(CPW // LANES):
        sl = pl.ds(i * LANES, LANES)
        ng_v[sl] = df_v[sl] - dm_v[sl]

    pltpu.sync_copy(dm_v, dm_o.at[pl.ds(base, CPW)])
    pltpu.sync_copy(df_v, df_o.at[pl.ds(base, CPW)])
    pltpu.sync_copy(ng_v, neg_o.at[pl.ds(base, CPW)])


@jax.jit
def _run(human_embeds, gmf_embeds, hm, rm, tm, hf, rf, tf):
    out = jax.ShapeDtypeStruct((BATCH,), jnp.float32)
    k = functools.partial(
        pl.kernel,
        out_type=[out, out, out],
        mesh=plsc.VectorSubcoreMesh(core_axis_name="c", subcore_axis_name="s"),
        compiler_params=pltpu.CompilerParams(
            needs_layout_passes=False, use_tc_tiling_on_sc=False),
        scratch_types=[
            pltpu.VMEM((CPW,), jnp.int32),
            pltpu.VMEM((CPW,), jnp.int32),
            pltpu.VMEM((CPW,), jnp.int32),
            pltpu.VMEM((CPW,), jnp.int32),
            pltpu.VMEM((CPW,), jnp.int32),
            pltpu.VMEM((CPW,), jnp.int32),
            pltpu.VMEM((CHUNK, DIM), jnp.float32),
            pltpu.VMEM((CHUNK, DIM), jnp.float32),
            pltpu.VMEM((CHUNK, DIM), jnp.float32),
            pltpu.VMEM((LANES * LANES,), jnp.float32),
            pltpu.VMEM((CPW,), jnp.float32),
            pltpu.VMEM((CPW,), jnp.float32),
            pltpu.VMEM((CPW,), jnp.float32),
            pltpu.SemaphoreType.DMA,
        ],
    )(_body)
    return k(human_embeds, gmf_embeds, hm, rm, tm, hf, rf, tf)


def kernel(human_embeds, gmf_embeds, male_triplets, female_triplets):
    hm = male_triplets[:, 0]
    rm = male_triplets[:, 1]
    tm = male_triplets[:, 2]
    hf = female_triplets[:, 0]
    rf = female_triplets[:, 1]
    tf = female_triplets[:, 2]
    neg, dm, df = _run(human_embeds, gmf_embeds, hm, rm, tm, hf, rf, tf)
    return (neg, dm, df)
